# initial kernel scaffold (unmeasured)
import jax
import jax.numpy as jnp
from jax import lax
from jax.experimental import pallas as pl
from jax.experimental.pallas import tpu as pltpu

N_DEV = 32
B_LOC = 2
SQ = 128
SKV = 128
H_LOC = 4
DH = 64
DM = 512
HD = H_LOC * DH
BLK = 64


def _body(x_ref, w_ref, k_ref, v_ref, out_ref, comm_ref, send_sems, recv_sems):
    h = pl.program_id(0)
    my = lax.axis_index("i")
    left = jnp.mod(my - 1, N_DEV)
    right = jnp.mod(my + 1, N_DEV)
    slot = lax.rem(h, 2)

    @pl.when(h == 0)
    def _():
        barrier = pltpu.get_barrier_semaphore()
        for nbr in (left, right):
            pl.semaphore_signal(
                barrier, inc=1,
                device_id=(nbr,), device_id_type=pl.DeviceIdType.MESH,
            )
        pl.semaphore_wait(barrier, 2)
        comm_ref[0] = w_ref[...]

    rdma = pltpu.make_async_remote_copy(
        src_ref=comm_ref.at[slot],
        dst_ref=comm_ref.at[1 - slot],
        send_sem=send_sems.at[slot],
        recv_sem=recv_sems.at[1 - slot],
        device_id=(right,),
        device_id_type=pl.DeviceIdType.MESH,
    )

    @pl.when(h < N_DEV - 1)
    def _():
        rdma.start()

    j = jnp.mod(my - h, N_DEV)
    col0 = j * HD

    wqT = comm_ref[slot, 0:HD, :]
    wo = comm_ref[slot, HD:2 * HD, :]

    x2 = x_ref[...].reshape(B_LOC * SQ, DM)
    q = lax.dot_general(
        x2, wqT, (((1,), (1,)), ((), ())),
        preferred_element_type=jnp.float32,
        precision=lax.Precision.HIGHEST,
    )

    qi = lax.broadcasted_iota(jnp.int32, (SQ, SKV), 0) // BLK
    kj = lax.broadcasted_iota(jnp.int32, (SQ, SKV), 1) // BLK
    mask = kj <= qi

    ctx_rows = []
    for b in range(B_LOC):
        kb = k_ref[b, :, pl.ds(col0, HD)]
        vb = v_ref[b, :, pl.ds(col0, HD)]
        row = []
        for hh in range(H_LOC):
            qbh = q[b * SQ:(b + 1) * SQ, hh * DH:(hh + 1) * DH]
            kbh = kb[:, hh * DH:(hh + 1) * DH]
            s = lax.dot_general(
                qbh, kbh, (((1,), (1,)), ((), ())),
                preferred_element_type=jnp.float32,
                precision=lax.Precision.HIGHEST,
            ) * 0.125
            s = jnp.where(mask, s, -1e9)
            m = jnp.max(s, axis=-1, keepdims=True)
            w = jnp.exp(s - m)
            w = w / jnp.sum(w, axis=-1, keepdims=True)
            vbh = vb[:, hh * DH:(hh + 1) * DH]
            row.append(jnp.dot(
                w, vbh,
                preferred_element_type=jnp.float32,
                precision=lax.Precision.HIGHEST,
            ))
        ctx_rows.append(jnp.concatenate(row, axis=1))
    ctx = jnp.concatenate(ctx_rows, axis=0)

    part = jnp.dot(
        ctx, wo,
        preferred_element_type=jnp.float32,
        precision=lax.Precision.HIGHEST,
    ).reshape(B_LOC, SQ, DM)

    @pl.when(h == 0)
    def _():
        out_ref[...] = part

    @pl.when(h > 0)
    def _():
        out_ref[...] = out_ref[...] + part

    @pl.when(h < N_DEV - 1)
    def _():
        rdma.wait()


def kernel(x, Wq, K_ext, V_ext, Wo):
    my = lax.axis_index("i")
    K_loc = lax.dynamic_slice_in_dim(K_ext, my * B_LOC, B_LOC, axis=0)
    V_loc = lax.dynamic_slice_in_dim(V_ext, my * B_LOC, B_LOC, axis=0)
    K2 = K_loc.reshape(B_LOC, SKV, N_DEV * HD)
    V2 = V_loc.reshape(B_LOC, SKV, N_DEV * HD)
    w_comb = jnp.concatenate([Wq.T, Wo], axis=0)

    return pl.pallas_call(
        _body,
        grid=(N_DEV,),
        out_shape=jax.ShapeDtypeStruct((B_LOC, SQ, DM), jnp.float32),
        in_specs=[pl.BlockSpec(memory_space=pltpu.VMEM)] * 4,
        out_specs=pl.BlockSpec(memory_space=pltpu.VMEM),
        scratch_shapes=[
            pltpu.VMEM((2, 2 * HD, DM), jnp.float32),
            pltpu.SemaphoreType.DMA((2,)),
            pltpu.SemaphoreType.DMA((2,)),
        ],
        compiler_params=pltpu.CompilerParams(
            dimension_semantics=("arbitrary",),
            collective_id=0,
        ),
    )(x, w_comb, K2, V2)


# baseline (device time: 271924 ns/iter reference)
import os

import jax
import jax.numpy as jnp
from jax import lax
from jax.experimental import pallas as pl
from jax.experimental.pallas import tpu as pltpu

_INTERPRET = os.environ.get("KERNEL_INTERPRET") == "1"

N_DEV = 32
B_LOC = 2
SQ = 128
SKV = 128
H_LOC = 4
DH = 64
DM = 512
HD = H_LOC * DH
BLK = 64

S = 4
T_F = 16
T_B = 15
GRID = T_F + 1


def _chunk_compute(x2_ref, k_ref, v_ref, comm_ref, slot, j, mask):
    wqT = comm_ref[slot, 0:HD, :]
    wo = comm_ref[slot, HD:2 * HD, :]

    q = lax.dot_general(
        x2_ref[...], wqT, (((1,), (1,)), ((), ())),
        preferred_element_type=jnp.float32,
    )

    col0 = j * HD
    ctx_rows = []
    for b in range(B_LOC):
        kb = k_ref[b, :, pl.ds(col0, HD)]
        vb = v_ref[b, :, pl.ds(col0, HD)]
        row = []
        for hh in range(H_LOC):
            qbh = q[b * SQ:(b + 1) * SQ, hh * DH:(hh + 1) * DH]
            kbh = kb[:, hh * DH:(hh + 1) * DH]
            s = lax.dot_general(
                qbh.astype(jnp.bfloat16), kbh, (((1,), (1,)), ((), ())),
                preferred_element_type=jnp.float32,
            ) * 0.125
            s = jnp.where(mask, s, -1e9)
            m = jnp.max(s, axis=-1, keepdims=True)
            w = jnp.exp(s - m)
            w = w / jnp.sum(w, axis=-1, keepdims=True)
            vbh = vb[:, hh * DH:(hh + 1) * DH]
            row.append(jnp.dot(
                w.astype(jnp.bfloat16), vbh,
                preferred_element_type=jnp.float32,
            ))
        ctx_rows.append(jnp.concatenate(row, axis=1))
    ctx = jnp.concatenate(ctx_rows, axis=0)

    part = jnp.dot(
        ctx.astype(jnp.bfloat16), wo,
        preferred_element_type=jnp.float32,
    )
    return part.reshape(B_LOC, SQ, DM)


def _body(x2_ref, w_ref, k_ref, v_ref, out_ref,
          fwd_comm, bwd_comm, fwd_send, fwd_recv, bwd_send, bwd_recv):
    t = pl.program_id(0)
    my = lax.axis_index("i")
    left = jnp.mod(my - 1, N_DEV)
    right = jnp.mod(my + 1, N_DEV)
    sslot = jnp.mod(t - 1, S)
    rslot = jnp.mod(t, S)

    qi = lax.broadcasted_iota(jnp.int32, (SQ, SKV), 0) // BLK
    kj = lax.broadcasted_iota(jnp.int32, (SQ, SKV), 1) // BLK
    mask = kj <= qi

    @pl.when(t == 0)
    def _():
        barrier = pltpu.get_barrier_semaphore()
        for nbr in (left, right):
            pl.semaphore_signal(
                barrier, inc=1,
                device_id=(nbr,), device_id_type=pl.DeviceIdType.MESH,
            )
        pl.semaphore_wait(barrier, 2)
        fwd_comm[0] = w_ref[...]
        bwd_comm[0] = w_ref[...]
        out_ref[...] = _chunk_compute(
            x2_ref, k_ref, v_ref, fwd_comm, 0, my, mask)

    fwd_rdma = pltpu.make_async_remote_copy(
        src_ref=fwd_comm.at[sslot],
        dst_ref=fwd_comm.at[rslot],
        send_sem=fwd_send.at[sslot],
        recv_sem=fwd_recv.at[rslot],
        device_id=(right,),
        device_id_type=pl.DeviceIdType.MESH,
    )
    bwd_rdma = pltpu.make_async_remote_copy(
        src_ref=bwd_comm.at[sslot],
        dst_ref=bwd_comm.at[rslot],
        send_sem=bwd_send.at[sslot],
        recv_sem=bwd_recv.at[rslot],
        device_id=(left,),
        device_id_type=pl.DeviceIdType.MESH,
    )

    @pl.when(t >= 1)
    def _():
        fwd_rdma.start()

    @pl.when((t >= 1) & (t <= T_B))
    def _():
        bwd_rdma.start()

    @pl.when(t >= 1)
    def _():
        fwd_rdma.wait()
        cf = jnp.mod(my - t, N_DEV)
        pf = _chunk_compute(x2_ref, k_ref, v_ref, fwd_comm, rslot, cf, mask)
        out_ref[...] = out_ref[...] + pf

    @pl.when((t >= 1) & (t <= T_B))
    def _():
        bwd_rdma.wait()
        cb = jnp.mod(my + t, N_DEV)
        pb = _chunk_compute(x2_ref, k_ref, v_ref, bwd_comm, rslot, cb, mask)
        out_ref[...] = out_ref[...] + pb


def kernel(x, Wq, K_ext, V_ext, Wo):
    my = lax.axis_index("i")
    K_loc = lax.dynamic_slice_in_dim(K_ext, my * B_LOC, B_LOC, axis=0)
    V_loc = lax.dynamic_slice_in_dim(V_ext, my * B_LOC, B_LOC, axis=0)
    K2 = K_loc.reshape(B_LOC, SKV, N_DEV * HD).astype(jnp.bfloat16)
    V2 = V_loc.reshape(B_LOC, SKV, N_DEV * HD).astype(jnp.bfloat16)
    x2 = x.reshape(B_LOC * SQ, DM).astype(jnp.bfloat16)
    w_comb = jnp.concatenate([Wq.T, Wo], axis=0).astype(jnp.bfloat16)

    return pl.pallas_call(
        _body,
        grid=(GRID,),
        out_shape=jax.ShapeDtypeStruct((B_LOC, SQ, DM), jnp.float32),
        in_specs=[pl.BlockSpec(memory_space=pltpu.VMEM)] * 4,
        out_specs=pl.BlockSpec(memory_space=pltpu.VMEM),
        scratch_shapes=[
            pltpu.VMEM((S, 2 * HD, DM), jnp.bfloat16),
            pltpu.VMEM((S, 2 * HD, DM), jnp.bfloat16),
            pltpu.SemaphoreType.DMA((S,)),
            pltpu.SemaphoreType.DMA((S,)),
            pltpu.SemaphoreType.DMA((S,)),
            pltpu.SemaphoreType.DMA((S,)),
        ],
        compiler_params=pltpu.CompilerParams(
            dimension_semantics=("arbitrary",),
            collective_id=0,
        ),
        interpret=(
            pltpu.InterpretParams(detect_races=True) if _INTERPRET else False
        ),
    )(x2, w_comb, K2, V2)


# device time: 238347 ns/iter; 1.1409x vs baseline; 1.1409x over previous
import os

import jax
import jax.numpy as jnp
from jax import lax
from jax.experimental import pallas as pl
from jax.experimental.pallas import tpu as pltpu

_INTERPRET = os.environ.get("KERNEL_INTERPRET") == "1"

N_DEV = 32
B_LOC = 2
SQ = 128
SKV = 128
H_LOC = 4
DH = 64
DM = 512
HD = H_LOC * DH
BLK = 64

S = 4
T_F = 16
T_B = 15
GRID = T_F + 2


def _chunk_compute(x2_ref, k_ref, v_ref, comm_ref, slot, j, maskf):
    wqT = comm_ref[slot, 0:HD, :]
    wo = comm_ref[slot, HD:2 * HD, :]

    q = lax.dot_general(
        x2_ref[...], wqT, (((1,), (1,)), ((), ())),
        preferred_element_type=jnp.float32,
    )

    col0 = j * HD
    ctx_rows = []
    for b in range(B_LOC):
        kb = k_ref[b, :, pl.ds(col0, HD)]
        vb = v_ref[b, :, pl.ds(col0, HD)]
        row = []
        for hh in range(H_LOC):
            qbh = q[b * SQ:(b + 1) * SQ, hh * DH:(hh + 1) * DH]
            kbh = kb[:, hh * DH:(hh + 1) * DH]
            s = lax.dot_general(
                qbh.astype(jnp.bfloat16), kbh, (((1,), (1,)), ((), ())),
                preferred_element_type=jnp.float32,
            ) * 0.125
            w = jnp.exp(s) * maskf
            w = w / jnp.sum(w, axis=-1, keepdims=True)
            vbh = vb[:, hh * DH:(hh + 1) * DH]
            row.append(jnp.dot(
                w.astype(jnp.bfloat16), vbh,
                preferred_element_type=jnp.float32,
            ))
        ctx_rows.append(jnp.concatenate(row, axis=1))
    ctx = jnp.concatenate(ctx_rows, axis=0)

    part = jnp.dot(
        ctx.astype(jnp.bfloat16), wo,
        preferred_element_type=jnp.float32,
    )
    return part.reshape(B_LOC, SQ, DM)


def _body(x2_ref, w_ref, k_ref, v_ref, out_ref,
          fwd_comm, bwd_comm, fwd_send, fwd_recv, bwd_send, bwd_recv):
    t = pl.program_id(0)
    my = lax.axis_index("i")
    left = jnp.mod(my - 1, N_DEV)
    right = jnp.mod(my + 1, N_DEV)
    sslot = jnp.mod(t - 1, S)
    rslot = jnp.mod(t, S)

    qi = lax.broadcasted_iota(jnp.int32, (SQ, SKV), 0) // BLK
    kj = lax.broadcasted_iota(jnp.int32, (SQ, SKV), 1) // BLK
    maskf = (kj <= qi).astype(jnp.float32)

    @pl.when(t == 0)
    def _():
        barrier = pltpu.get_barrier_semaphore()
        for nbr in (left, right):
            pl.semaphore_signal(
                barrier, inc=1,
                device_id=(nbr,), device_id_type=pl.DeviceIdType.MESH,
            )
        pl.semaphore_wait(barrier, 2)
        fwd_comm[0] = w_ref[...]
        bwd_comm[0] = w_ref[...]

    fwd_rdma = pltpu.make_async_remote_copy(
        src_ref=fwd_comm.at[sslot],
        dst_ref=fwd_comm.at[rslot],
        send_sem=fwd_send.at[sslot],
        recv_sem=fwd_recv.at[rslot],
        device_id=(right,),
        device_id_type=pl.DeviceIdType.MESH,
    )
    bwd_rdma = pltpu.make_async_remote_copy(
        src_ref=bwd_comm.at[sslot],
        dst_ref=bwd_comm.at[rslot],
        send_sem=bwd_send.at[sslot],
        recv_sem=bwd_recv.at[rslot],
        device_id=(left,),
        device_id_type=pl.DeviceIdType.MESH,
    )

    @pl.when((t >= 1) & (t <= T_F))
    def _():
        fwd_rdma.start()

    @pl.when((t >= 1) & (t <= T_B))
    def _():
        bwd_rdma.start()

    @pl.when(t >= 1)
    def _():
        cf = jnp.mod(my - (t - 1), N_DEV)
        pf = _chunk_compute(x2_ref, k_ref, v_ref, fwd_comm, sslot, cf, maskf)

        @pl.when(t == 1)
        def _():
            out_ref[...] = pf

        @pl.when(t > 1)
        def _():
            out_ref[...] = out_ref[...] + pf

    @pl.when((t >= 2) & (t <= T_B + 1))
    def _():
        cb = jnp.mod(my + (t - 1), N_DEV)
        pb = _chunk_compute(x2_ref, k_ref, v_ref, bwd_comm, sslot, cb, maskf)
        out_ref[...] = out_ref[...] + pb

    @pl.when((t >= 1) & (t <= T_F))
    def _():
        fwd_rdma.wait_recv()
        fwd_rdma.wait_send()

    @pl.when((t >= 1) & (t <= T_B))
    def _():
        bwd_rdma.wait_recv()
        bwd_rdma.wait_send()


def kernel(x, Wq, K_ext, V_ext, Wo):
    my = lax.axis_index("i")
    K_loc = lax.dynamic_slice_in_dim(K_ext, my * B_LOC, B_LOC, axis=0)
    V_loc = lax.dynamic_slice_in_dim(V_ext, my * B_LOC, B_LOC, axis=0)
    K2 = K_loc.reshape(B_LOC, SKV, N_DEV * HD).astype(jnp.bfloat16)
    V2 = V_loc.reshape(B_LOC, SKV, N_DEV * HD).astype(jnp.bfloat16)
    x2 = x.reshape(B_LOC * SQ, DM).astype(jnp.bfloat16)
    w_comb = jnp.concatenate([Wq.T, Wo], axis=0).astype(jnp.bfloat16)

    return pl.pallas_call(
        _body,
        grid=(GRID,),
        out_shape=jax.ShapeDtypeStruct((B_LOC, SQ, DM), jnp.float32),
        in_specs=[pl.BlockSpec(memory_space=pltpu.VMEM)] * 4,
        out_specs=pl.BlockSpec(memory_space=pltpu.VMEM),
        scratch_shapes=[
            pltpu.VMEM((S, 2 * HD, DM), jnp.bfloat16),
            pltpu.VMEM((S, 2 * HD, DM), jnp.bfloat16),
            pltpu.SemaphoreType.DMA((S,)),
            pltpu.SemaphoreType.DMA((S,)),
            pltpu.SemaphoreType.DMA((S,)),
            pltpu.SemaphoreType.DMA((S,)),
        ],
        compiler_params=pltpu.CompilerParams(
            dimension_semantics=("arbitrary",),
            collective_id=0,
        ),
        interpret=(
            pltpu.InterpretParams(detect_races=True) if _INTERPRET else False
        ),
    )(x2, w_comb, K2, V2)
